# Initial kernel scaffold; baseline (speedup 1.0000x reference)
import jax
import jax.numpy as jnp
from jax.experimental import pallas as pl


def kernel(score, mask):
    raise NotImplementedError("write your pallas kernel here")


# speedup vs baseline: 5.3203x; 5.3203x over previous
import functools

import jax
import jax.numpy as jnp
from jax.experimental import pallas as pl
from jax.experimental.pallas import tpu as pltpu

_THRESHOLD = 0.5
_INF = float("inf")


def _shift_rows(a, k, fill):
    if k > 0:
        pad = jnp.full((k, a.shape[1]), fill, a.dtype)
        return jnp.concatenate([a[k:], pad], axis=0)
    if k < 0:
        pad = jnp.full((-k, a.shape[1]), fill, a.dtype)
        return jnp.concatenate([pad, a[:k]], axis=0)
    return a


def _shift_cols(a, k, fill):
    if k > 0:
        pad = jnp.full((a.shape[0], k), fill, a.dtype)
        return jnp.concatenate([a[:, k:], pad], axis=1)
    if k < 0:
        pad = jnp.full((a.shape[0], -k), fill, a.dtype)
        return jnp.concatenate([pad, a[:, :k]], axis=1)
    return a


def _mask_kernel(score_ref, mask_ref, score_out_ref, mask_out_ref):
    s = score_ref[0]
    score_out_ref[0] = jnp.where(s >= _THRESHOLD, s, jnp.zeros((), s.dtype))

    m = mask_ref[0]
    t = jnp.where(m >= _THRESHOLD, m, jnp.zeros((), m.dtype))

    r = jnp.minimum(
        jnp.minimum(_shift_rows(t, -2, _INF), _shift_rows(t, -1, _INF)),
        jnp.minimum(t, _shift_rows(t, 1, _INF)),
    )
    e = jnp.minimum(
        jnp.minimum(_shift_cols(r, -2, _INF), _shift_cols(r, -1, _INF)),
        jnp.minimum(r, _shift_cols(r, 1, _INF)),
    )

    r = jnp.maximum(
        jnp.maximum(_shift_rows(e, -1, -_INF), e),
        jnp.maximum(_shift_rows(e, 1, -_INF), _shift_rows(e, 2, -_INF)),
    )
    d = jnp.maximum(
        jnp.maximum(_shift_cols(r, -1, -_INF), r),
        jnp.maximum(_shift_cols(r, 1, -_INF), _shift_cols(r, 2, -_INF)),
    )
    mask_out_ref[0] = d


@jax.jit
def kernel(score, mask):
    b, n = score.shape
    score3 = score.reshape(b, 1, n)
    score_out, mask_out = pl.pallas_call(
        _mask_kernel,
        grid=(b,),
        in_specs=[
            pl.BlockSpec((1, 1, n), lambda i: (i, 0, 0)),
            pl.BlockSpec((1, mask.shape[1], mask.shape[2]), lambda i: (i, 0, 0)),
        ],
        out_specs=[
            pl.BlockSpec((1, 1, n), lambda i: (i, 0, 0)),
            pl.BlockSpec((1, mask.shape[1], mask.shape[2]), lambda i: (i, 0, 0)),
        ],
        out_shape=[
            jax.ShapeDtypeStruct(score3.shape, score.dtype),
            jax.ShapeDtypeStruct(mask.shape, mask.dtype),
        ],
        compiler_params=pltpu.CompilerParams(
            dimension_semantics=("parallel",),
            vmem_limit_bytes=100 * 1024 * 1024,
        ),
    )(score3, mask)
    return (score_out.reshape(b, n), mask_out)


# speedup vs baseline: 6.6822x; 1.2560x over previous
import functools

import jax
import jax.numpy as jnp
from jax.experimental import pallas as pl
from jax.experimental.pallas import tpu as pltpu

_THRESHOLD = 0.5
_INF = float("inf")


def _shift_rows(a, k, fill):
    if k > 0:
        pad = jnp.full((k, a.shape[1]), fill, a.dtype)
        return jnp.concatenate([a[k:], pad], axis=0)
    if k < 0:
        pad = jnp.full((-k, a.shape[1]), fill, a.dtype)
        return jnp.concatenate([pad, a[:k]], axis=0)
    return a


def _shift_cols(a, k, fill):
    if k > 0:
        pad = jnp.full((a.shape[0], k), fill, a.dtype)
        return jnp.concatenate([a[:, k:], pad], axis=1)
    if k < 0:
        pad = jnp.full((a.shape[0], -k), fill, a.dtype)
        return jnp.concatenate([pad, a[:, :k]], axis=1)
    return a


def _mask_kernel(score_ref, mask_ref, score_out_ref, mask_out_ref):
    s = score_ref[0]
    score_out_ref[0] = jnp.where(s >= _THRESHOLD, s, jnp.zeros((), s.dtype))

    m = mask_ref[0]
    t = jnp.where(m >= _THRESHOLD, m, jnp.zeros((), m.dtype))

    def erode_rows(t):
        p = jnp.minimum(t, _shift_rows(t, 1, _INF))
        inf_row = jnp.full((1, t.shape[1]), _INF, t.dtype)
        sh = jnp.concatenate([inf_row, t[0:1], p[:-2]], axis=0)
        return jnp.minimum(sh, p)

    def erode_cols(t):
        p = jnp.minimum(t, _shift_cols(t, 1, _INF))
        inf_col = jnp.full((t.shape[0], 1), _INF, t.dtype)
        sh = jnp.concatenate([inf_col, t[:, 0:1], p[:, :-2]], axis=1)
        return jnp.minimum(sh, p)

    def dilate_rows(e):
        q = jnp.maximum(e, _shift_rows(e, 1, -_INF))
        lo = jnp.concatenate([e[0:1], q[:-1]], axis=0)
        return jnp.maximum(lo, _shift_rows(q, 1, -_INF))

    def dilate_cols(e):
        q = jnp.maximum(e, _shift_cols(e, 1, -_INF))
        lo = jnp.concatenate([e[:, 0:1], q[:, :-1]], axis=1)
        return jnp.maximum(lo, _shift_cols(q, 1, -_INF))

    e = erode_cols(erode_rows(t))
    mask_out_ref[0] = dilate_cols(dilate_rows(e))


@jax.jit
def kernel(score, mask):
    b, n = score.shape
    score3 = score.reshape(b, 1, n)
    score_out, mask_out = pl.pallas_call(
        _mask_kernel,
        grid=(b,),
        in_specs=[
            pl.BlockSpec((1, 1, n), lambda i: (i, 0, 0)),
            pl.BlockSpec((1, mask.shape[1], mask.shape[2]), lambda i: (i, 0, 0)),
        ],
        out_specs=[
            pl.BlockSpec((1, 1, n), lambda i: (i, 0, 0)),
            pl.BlockSpec((1, mask.shape[1], mask.shape[2]), lambda i: (i, 0, 0)),
        ],
        out_shape=[
            jax.ShapeDtypeStruct(score3.shape, score.dtype),
            jax.ShapeDtypeStruct(mask.shape, mask.dtype),
        ],
        compiler_params=pltpu.CompilerParams(
            dimension_semantics=("parallel",),
            vmem_limit_bytes=100 * 1024 * 1024,
        ),
    )(score3, mask)
    return (score_out.reshape(b, n), mask_out)


# speedup vs baseline: 7.2324x; 1.0823x over previous
import jax
import jax.numpy as jnp
from jax.experimental import pallas as pl
from jax.experimental.pallas import tpu as pltpu

_THRESHOLD = 0.5
_INF = float("inf")
_N = 1024
_STRIP = 64
_NSTRIP = _N // _STRIP


def _shift_cols(a, k, fill):
    if k > 0:
        pad = jnp.full((a.shape[0], k), fill, a.dtype)
        return jnp.concatenate([a[:, k:], pad], axis=1)
    if k < 0:
        pad = jnp.full((a.shape[0], -k), fill, a.dtype)
        return jnp.concatenate([pad, a[:, :k]], axis=1)
    return a


def _erode_cols(x):
    p = jnp.minimum(x, _shift_cols(x, 1, _INF))
    inf_col = jnp.full((x.shape[0], 1), _INF, x.dtype)
    sh = jnp.concatenate([inf_col, x[:, 0:1], p[:, :-2]], axis=1)
    return jnp.minimum(sh, p)


def _dilate_cols(x):
    q = jnp.maximum(_shift_cols(x, -1, -_INF), x)
    ninf_col = jnp.full((x.shape[0], 1), -_INF, x.dtype)
    sh = jnp.concatenate([q[:, 2:], x[:, -1:], ninf_col], axis=1)
    return jnp.maximum(q, sh)


def _mask_kernel(score_ref, mask_ref, score_out_ref, mask_out_ref):
    s = score_ref[0]
    score_out_ref[0] = jnp.where(s >= _THRESHOLD, s, jnp.zeros((), s.dtype))

    inf_rows8 = jnp.full((8, _N), _INF, jnp.float32)
    ninf_row = jnp.full((1, _N), -_INF, jnp.float32)

    for st in range(_NSTRIP):
        r0 = st * _STRIP
        if st == 0:
            a = jnp.concatenate([inf_rows8, mask_ref[0, 0:_STRIP + 8, :]], axis=0)
        elif st == _NSTRIP - 1:
            a = jnp.concatenate([mask_ref[0, r0 - 8:_N, :], inf_rows8], axis=0)
        else:
            a = mask_ref[0, r0 - 8:r0 + _STRIP + 8, :]

        p = jnp.minimum(a[5:5 + _STRIP + 6, :], a[6:6 + _STRIP + 6, :])
        er = jnp.minimum(p[0:_STRIP + 4, :], p[2:2 + _STRIP + 4, :])

        ec = _erode_cols(er)
        et = jnp.where(ec >= _THRESHOLD, ec, jnp.zeros((), ec.dtype))

        if st == 0:
            et = jnp.concatenate([ninf_row, et[1:, :]], axis=0)
        elif st == _NSTRIP - 1:
            et = jnp.concatenate([et[0:_STRIP + 1, :],
                                  jnp.full((3, _N), -_INF, jnp.float32)], axis=0)

        q = jnp.maximum(et[0:_STRIP + 2, :], et[1:_STRIP + 3, :])
        dr = jnp.maximum(q[0:_STRIP, :], q[2:_STRIP + 2, :])

        mask_out_ref[0, r0:r0 + _STRIP, :] = _dilate_cols(dr)


@jax.jit
def kernel(score, mask):
    b, n = score.shape
    score3 = score.reshape(b, 1, n)
    score_out, mask_out = pl.pallas_call(
        _mask_kernel,
        grid=(b,),
        in_specs=[
            pl.BlockSpec((1, 1, n), lambda i: (i, 0, 0)),
            pl.BlockSpec((1, _N, _N), lambda i: (i, 0, 0)),
        ],
        out_specs=[
            pl.BlockSpec((1, 1, n), lambda i: (i, 0, 0)),
            pl.BlockSpec((1, _N, _N), lambda i: (i, 0, 0)),
        ],
        out_shape=[
            jax.ShapeDtypeStruct(score3.shape, score.dtype),
            jax.ShapeDtypeStruct(mask.shape, mask.dtype),
        ],
        compiler_params=pltpu.CompilerParams(
            dimension_semantics=("arbitrary",),
            vmem_limit_bytes=100 * 1024 * 1024,
        ),
    )(score3, mask)
    return (score_out.reshape(b, n), mask_out)


# speedup vs baseline: 8.3490x; 1.1544x over previous
import jax
import jax.numpy as jnp
from jax.experimental import pallas as pl
from jax.experimental.pallas import tpu as pltpu

_THRESHOLD = 0.5
_INF = float("inf")
_N = 1024
_STRIP = 256
_NSTRIP = _N // _STRIP


def _shift_cols(a, k, fill):
    if k > 0:
        pad = jnp.full((a.shape[0], k), fill, a.dtype)
        return jnp.concatenate([a[:, k:], pad], axis=1)
    if k < 0:
        pad = jnp.full((a.shape[0], -k), fill, a.dtype)
        return jnp.concatenate([pad, a[:, :k]], axis=1)
    return a


def _erode_cols(x):
    p = jnp.minimum(x, _shift_cols(x, 1, _INF))
    inf_col = jnp.full((x.shape[0], 1), _INF, x.dtype)
    sh = jnp.concatenate([inf_col, x[:, 0:1], p[:, :-2]], axis=1)
    return jnp.minimum(sh, p)


def _dilate_cols(x):
    q = jnp.maximum(_shift_cols(x, -1, -_INF), x)
    ninf_col = jnp.full((x.shape[0], 1), -_INF, x.dtype)
    sh = jnp.concatenate([q[:, 2:], x[:, -1:], ninf_col], axis=1)
    return jnp.maximum(q, sh)


def _mask_kernel(score_ref, mask_ref, score_out_ref, mask_out_ref):
    s = score_ref[0]
    score_out_ref[0] = jnp.where(s >= _THRESHOLD, s, jnp.zeros((), s.dtype))

    inf_rows8 = jnp.full((8, _N), _INF, jnp.float32)
    ninf_row = jnp.full((1, _N), -_INF, jnp.float32)

    for st in range(_NSTRIP):
        r0 = st * _STRIP
        if st == 0:
            a = jnp.concatenate([inf_rows8, mask_ref[0, 0:_STRIP + 8, :]], axis=0)
        elif st == _NSTRIP - 1:
            a = jnp.concatenate([mask_ref[0, r0 - 8:_N, :], inf_rows8], axis=0)
        else:
            a = mask_ref[0, r0 - 8:r0 + _STRIP + 8, :]

        p = jnp.minimum(a[5:5 + _STRIP + 6, :], a[6:6 + _STRIP + 6, :])
        er = jnp.minimum(p[0:_STRIP + 4, :], p[2:2 + _STRIP + 4, :])

        ec = _erode_cols(er)
        et = jnp.where(ec >= _THRESHOLD, ec, jnp.zeros((), ec.dtype))

        if st == 0:
            et = jnp.concatenate([ninf_row, et[1:, :]], axis=0)
        elif st == _NSTRIP - 1:
            et = jnp.concatenate([et[0:_STRIP + 1, :],
                                  jnp.full((3, _N), -_INF, jnp.float32)], axis=0)

        q = jnp.maximum(et[0:_STRIP + 2, :], et[1:_STRIP + 3, :])
        dr = jnp.maximum(q[0:_STRIP, :], q[2:_STRIP + 2, :])

        mask_out_ref[0, r0:r0 + _STRIP, :] = _dilate_cols(dr)


@jax.jit
def kernel(score, mask):
    b, n = score.shape
    score3 = score.reshape(b, 1, n)
    score_out, mask_out = pl.pallas_call(
        _mask_kernel,
        grid=(b,),
        in_specs=[
            pl.BlockSpec((1, 1, n), lambda i: (i, 0, 0)),
            pl.BlockSpec((1, _N, _N), lambda i: (i, 0, 0)),
        ],
        out_specs=[
            pl.BlockSpec((1, 1, n), lambda i: (i, 0, 0)),
            pl.BlockSpec((1, _N, _N), lambda i: (i, 0, 0)),
        ],
        out_shape=[
            jax.ShapeDtypeStruct(score3.shape, score.dtype),
            jax.ShapeDtypeStruct(mask.shape, mask.dtype),
        ],
        compiler_params=pltpu.CompilerParams(
            dimension_semantics=("arbitrary",),
            vmem_limit_bytes=100 * 1024 * 1024,
        ),
    )(score3, mask)
    return (score_out.reshape(b, n), mask_out)
